# baseline (speedup 1.0000x reference)
import functools
import jax
import jax.numpy as jnp
from jax import lax
from jax.experimental import pallas as pl
from jax.experimental.pallas import tpu as pltpu
from jax.experimental.pallas import tpu_sc as plsc

BATCH = 16384
EMBED_DIM = 64
ROWS = 1000001

_NC = 2
_NS = 16
_L = 16
_NW = _NC * _NS

_REM1 = (ROWS // 128) * 128
_WT = 245
_WW = _WT * 128
_WWL = _WW - 256
_LOMAX = _REM1 - _WWL
_CAP = 768
_SENT = BATCH
_STG = (BATCH + 1) * 128


def _gather_body(uids, iids, utab, itab, utail, itail, ug, ig,
                 idbuf, cr_u, cb_u, cr_i, cb_i, slab_a, slab_b, rows,
                 sem_a, sem_b, esem):
    cid = lax.axis_index("c")
    sid = lax.axis_index("s")
    wid = sid * _NC + cid
    lanes = lax.iota(jnp.int32, _L)
    lo = jnp.minimum(wid * _WW, _LOMAX)
    hi = lo + _WW
    last = wid == _NW - 1

    def extract(ids_hbm, cr, cb):
        def chunk(c, ou):
            pltpu.sync_copy(
                ids_hbm.at[pl.ds(pl.multiple_of(c * 1024, 1024), 1024)],
                idbuf)

            def vec(i, ou):
                r = idbuf[pl.ds(pl.multiple_of(i * _L, _L), _L)] + 1
                b = c * 1024 + i * _L + lanes
                m = (r >= lo) & (r < hi)
                cnt = plsc.all_reduce_population_count(m)[0]
                plsc.store_compressed(cr.at[pl.ds(ou, _L)], r - lo, mask=m)
                plsc.store_compressed(cb.at[pl.ds(ou, _L)], b, mask=m)
                return jnp.minimum(ou + cnt, _CAP)

            return lax.fori_loop(0, 1024 // _L, vec, ou)

        ou = lax.fori_loop(0, BATCH // 1024, chunk, jnp.int32(0))
        full = jnp.ones((_L,), jnp.bool_)
        plsc.store_compressed(cr.at[pl.ds(ou, _L)],
                              jnp.zeros((_L,), jnp.int32), mask=full)
        plsc.store_compressed(cb.at[pl.ds(ou, _L)],
                              jnp.full((_L,), _SENT, jnp.int32), mask=full)
        return (ou + _L - 1) // _L

    ng_u = extract(uids, cr_u, cb_u)
    ng_i = extract(iids, cr_i, cb_i)

    def stage(tab, tail, slab, d, s):
        @pl.when(jnp.logical_not(last))
        def _main():
            pltpu.async_copy(tab.at[d, pl.ds(pl.multiple_of(lo, 128), _WW)],
                             slab, s)

        @pl.when(last)
        def _last():
            pltpu.async_copy(
                tab.at[d, pl.ds(pl.multiple_of(lo, 128), _WWL)],
                slab.at[pl.ds(0, _WWL)], s)
            pltpu.async_copy(tail.at[d], slab.at[pl.ds(_WWL, 128)], s)

    def stage_wait(tab, tail, slab, d, s):
        @pl.when(jnp.logical_not(last))
        def _main():
            pltpu.make_async_copy(
                tab.at[d, pl.ds(pl.multiple_of(lo, 128), _WW)],
                slab, s).wait()

        @pl.when(last)
        def _last():
            pltpu.make_async_copy(
                tab.at[d, pl.ds(pl.multiple_of(lo, 128), _WWL)],
                slab.at[pl.ds(0, _WWL)], s).wait()
            pltpu.make_async_copy(tail.at[d], slab.at[pl.ds(_WWL, 128)],
                                  s).wait()

    def table_phase(tab, tail, cr, cb, ng, out_hbm):
        def gather_d(slab, d):
            def grp(g, _):
                rl = cr[pl.ds(pl.multiple_of(g * _L, _L), _L)]
                vals = plsc.load_gather(slab, [rl])
                plsc.store_scatter(rows,
                                   [(g * _L + lanes) * EMBED_DIM + d], vals)
                return 0

            lax.fori_loop(0, ng, grp, 0)

        stage(tab, tail, slab_a, 0, sem_a)

        def pair(dp, _):
            d0 = dp * 2
            stage_wait(tab, tail, slab_a, d0, sem_a)
            stage(tab, tail, slab_b, d0 + 1, sem_b)
            gather_d(slab_a, d0)
            stage_wait(tab, tail, slab_b, d0 + 1, sem_b)

            @pl.when(dp + 1 < EMBED_DIM // 2)
            def _pf():
                stage(tab, tail, slab_a, d0 + 2, sem_a)

            gather_d(slab_b, d0 + 1)
            return 0

        lax.fori_loop(0, EMBED_DIM // 2, pair, 0)

        def egrp(g, _):
            bv = cb[pl.ds(pl.multiple_of(g * _L, _L), _L)]
            for l in range(_L):
                k = g * _L + l
                src = rows.at[pl.ds(k * EMBED_DIM, EMBED_DIM)]
                dst = out_hbm.at[pl.ds(pl.multiple_of(bv[l] * 128, 128),
                                       EMBED_DIM)]
                pltpu.async_copy(src, dst, esem)
            return 0

        lax.fori_loop(0, ng, egrp, 0)

        def edrain(k, _):
            pltpu.make_async_copy(
                rows.at[pl.ds(0, EMBED_DIM)],
                out_hbm.at[pl.ds(_SENT * 128, EMBED_DIM)], esem).wait()
            return 0

        lax.fori_loop(0, ng * _L, edrain, 0)

    table_phase(utab, utail, cr_u, cb_u, ng_u, ug)
    table_phase(itab, itail, cr_i, cb_i, ng_i, ig)


_BPT2 = BATCH // _NW
_CH2 = 256


def _dot_body(ug, ig, out, us, vs, outv, sem):
    cid = lax.axis_index("c")
    sid = lax.axis_index("s")
    wid = sid * _NC + cid
    base = wid * _BPT2
    lanes = lax.iota(jnp.int32, _L)

    for c in range(_BPT2 // _CH2):
        b0 = pl.multiple_of((base + c * _CH2) * 128, 128)
        pltpu.sync_copy(ug.at[pl.ds(b0, _CH2 * 128)], us)
        pltpu.sync_copy(ig.at[pl.ds(b0, _CH2 * 128)], vs)

        def grp(g, _):
            flat = (g * _L + lanes) * 128
            acc = jnp.zeros((_L,), jnp.float32)
            for d in range(EMBED_DIM):
                u = plsc.load_gather(us, [flat + d])
                v = plsc.load_gather(vs, [flat + d])
                acc = acc + u * v
            outv[pl.ds(pl.multiple_of(c * _CH2, _L) + g * _L, _L)] = acc
            return 0

        lax.fori_loop(0, _CH2 // _L, grp, 0)

    pltpu.sync_copy(outv, out.at[pl.ds(pl.multiple_of(base, 8), _BPT2)])


@jax.jit
def kernel(user_ids, item_ids, user_table, item_table):
    mesh = plsc.VectorSubcoreMesh(
        core_axis_name="c", subcore_axis_name="s",
        num_cores=_NC, num_subcores=_NS)
    gather_run = pl.kernel(
        _gather_body,
        out_type=(jax.ShapeDtypeStruct((_STG,), jnp.float32),
                  jax.ShapeDtypeStruct((_STG,), jnp.float32)),
        mesh=mesh,
        scratch_types=[
            pltpu.VMEM((1024,), jnp.int32),
            pltpu.VMEM((_CAP + _L,), jnp.int32),
            pltpu.VMEM((_CAP + _L,), jnp.int32),
            pltpu.VMEM((_CAP + _L,), jnp.int32),
            pltpu.VMEM((_CAP + _L,), jnp.int32),
            pltpu.VMEM((_WW,), jnp.float32),
            pltpu.VMEM((_WW,), jnp.float32),
            pltpu.VMEM((_CAP * EMBED_DIM,), jnp.float32),
            pltpu.SemaphoreType.DMA,
            pltpu.SemaphoreType.DMA,
            pltpu.SemaphoreType.DMA,
        ],
        compiler_params=pltpu.CompilerParams(needs_layout_passes=False),
    )
    dot_run = pl.kernel(
        _dot_body,
        out_type=jax.ShapeDtypeStruct((BATCH,), jnp.float32),
        mesh=mesh,
        scratch_types=[
            pltpu.VMEM((_CH2 * 128,), jnp.float32),
            pltpu.VMEM((_CH2 * 128,), jnp.float32),
            pltpu.VMEM((_BPT2,), jnp.float32),
            pltpu.SemaphoreType.DMA,
        ],
        compiler_params=pltpu.CompilerParams(needs_layout_passes=False),
    )
    utail = jnp.pad(user_table[_REM1:], ((0, 128 - (ROWS - _REM1)), (0, 0))).T
    itail = jnp.pad(item_table[_REM1:], ((0, 128 - (ROWS - _REM1)), (0, 0))).T
    ug, ig = gather_run(user_ids, item_ids, user_table.T, item_table.T,
                        utail, itail)
    return dot_run(ug, ig)


# speedup vs baseline: 1.1061x; 1.1061x over previous
import functools
import jax
import jax.numpy as jnp
from jax import lax
from jax.experimental import pallas as pl
from jax.experimental.pallas import tpu as pltpu
from jax.experimental.pallas import tpu_sc as plsc

BATCH = 16384
EMBED_DIM = 64
ROWS = 1000001

_NC = 2
_NS = 16
_L = 16
_NW = _NC * _NS

_REM1 = (ROWS // 128) * 128
_WT = 245
_WW = _WT * 128
_WWL = _WW - 256
_LOMAX = _REM1 - _WWL
_CAP = 768
_SENT = BATCH
_STRIDE = 72
_STG = (BATCH + 1) * _STRIDE


def _gather_body(uids, iids, utab, itab, utail, itail, ug, ig,
                 idbuf, cr_u, cb_u, cr_i, cb_i, slab_a, slab_b, rows,
                 sem_a, sem_b, esem):
    cid = lax.axis_index("c")
    sid = lax.axis_index("s")
    wid = sid * _NC + cid
    lanes = lax.iota(jnp.int32, _L)
    lo = jnp.minimum(wid * _WW, _LOMAX)
    hi = lo + _WW
    last = wid == _NW - 1

    def extract(ids_hbm, cr, cb):
        def chunk(c, ou):
            pltpu.sync_copy(
                ids_hbm.at[pl.ds(pl.multiple_of(c * 4096, 4096), 4096)],
                idbuf)

            def vec(i, ou):
                r = idbuf[pl.ds(pl.multiple_of(i * _L, _L), _L)] + 1
                b = c * 4096 + i * _L + lanes
                m = (r >= lo) & (r < hi)
                cnt = plsc.all_reduce_population_count(m)[0]
                plsc.store_compressed(cr.at[pl.ds(ou, _L)], r - lo, mask=m)
                plsc.store_compressed(cb.at[pl.ds(ou, _L)], b, mask=m)
                return jnp.minimum(ou + cnt, _CAP)

            return lax.fori_loop(0, 4096 // _L, vec, ou)

        ou = lax.fori_loop(0, BATCH // 4096, chunk, jnp.int32(0))
        full = jnp.ones((_L,), jnp.bool_)
        plsc.store_compressed(cr.at[pl.ds(ou, _L)],
                              jnp.zeros((_L,), jnp.int32), mask=full)
        plsc.store_compressed(cb.at[pl.ds(ou, _L)],
                              jnp.full((_L,), _SENT, jnp.int32), mask=full)
        return (ou + _L - 1) // _L

    ng_u = extract(uids, cr_u, cb_u)
    ng_i = extract(iids, cr_i, cb_i)

    def stage(tab, tail, slab, d, s):
        @pl.when(jnp.logical_not(last))
        def _main():
            pltpu.async_copy(tab.at[d, pl.ds(pl.multiple_of(lo, 128), _WW)],
                             slab, s)

        @pl.when(last)
        def _last():
            pltpu.async_copy(
                tab.at[d, pl.ds(pl.multiple_of(lo, 128), _WWL)],
                slab.at[pl.ds(0, _WWL)], s)
            pltpu.async_copy(tail.at[d], slab.at[pl.ds(_WWL, 128)], s)

    def stage_wait(tab, tail, slab, d, s):
        @pl.when(jnp.logical_not(last))
        def _main():
            pltpu.make_async_copy(
                tab.at[d, pl.ds(pl.multiple_of(lo, 128), _WW)],
                slab, s).wait()

        @pl.when(last)
        def _last():
            pltpu.make_async_copy(
                tab.at[d, pl.ds(pl.multiple_of(lo, 128), _WWL)],
                slab.at[pl.ds(0, _WWL)], s).wait()
            pltpu.make_async_copy(tail.at[d], slab.at[pl.ds(_WWL, 128)],
                                  s).wait()

    def table_phase(tab, tail, cr, cb, ng, out_hbm):
        def gather_d(slab, d):
            def grp(g, _):
                rl = cr[pl.ds(pl.multiple_of(g * _L, _L), _L)]
                vals = plsc.load_gather(slab, [rl])
                plsc.store_scatter(rows,
                                   [(g * _L + lanes) * _STRIDE + d], vals)
                return 0

            lax.fori_loop(0, ng, grp, 0)

        stage(tab, tail, slab_a, 0, sem_a)

        def pair(dp, _):
            d0 = dp * 2
            stage_wait(tab, tail, slab_a, d0, sem_a)
            stage(tab, tail, slab_b, d0 + 1, sem_b)
            gather_d(slab_a, d0)
            stage_wait(tab, tail, slab_b, d0 + 1, sem_b)

            @pl.when(dp + 1 < EMBED_DIM // 2)
            def _pf():
                stage(tab, tail, slab_a, d0 + 2, sem_a)

            gather_d(slab_b, d0 + 1)
            return 0

        lax.fori_loop(0, EMBED_DIM // 2, pair, 0)

        def egrp(g, _):
            bv = cb[pl.ds(pl.multiple_of(g * _L, _L), _L)]
            for l in range(_L):
                k = g * _L + l
                src = rows.at[pl.ds(k * _STRIDE, EMBED_DIM)]
                dst = out_hbm.at[pl.ds(pl.multiple_of(bv[l] * _STRIDE, 8),
                                       EMBED_DIM)]
                pltpu.async_copy(src, dst, esem)
            return 0

        lax.fori_loop(0, ng, egrp, 0)

        def edrain(k, _):
            pltpu.make_async_copy(
                rows.at[pl.ds(0, EMBED_DIM)],
                out_hbm.at[pl.ds(_SENT * _STRIDE, EMBED_DIM)], esem).wait()
            return 0

        lax.fori_loop(0, ng * _L, edrain, 0)

    table_phase(utab, utail, cr_u, cb_u, ng_u, ug)
    table_phase(itab, itail, cr_i, cb_i, ng_i, ig)


_BPT2 = BATCH // _NW
_CH2 = 256


def _dot_body(ug, ig, out, us, vs, outv, sem):
    cid = lax.axis_index("c")
    sid = lax.axis_index("s")
    wid = sid * _NC + cid
    base = wid * _BPT2
    lanes = lax.iota(jnp.int32, _L)

    b0 = pl.multiple_of(base * _STRIDE, 8)
    pltpu.sync_copy(ug.at[pl.ds(b0, _BPT2 * _STRIDE)], us)
    pltpu.sync_copy(ig.at[pl.ds(b0, _BPT2 * _STRIDE)], vs)

    def grp(g, _):
        flat = (g * _L + lanes) * _STRIDE
        acc = jnp.zeros((_L,), jnp.float32)
        for d in range(EMBED_DIM):
            u = plsc.load_gather(us, [flat + d])
            v = plsc.load_gather(vs, [flat + d])
            acc = acc + u * v
        outv[pl.ds(pl.multiple_of(g * _L, _L), _L)] = acc
        return 0

    lax.fori_loop(0, _BPT2 // _L, grp, 0)

    pltpu.sync_copy(outv, out.at[pl.ds(pl.multiple_of(base, 8), _BPT2)])


@jax.jit
def kernel(user_ids, item_ids, user_table, item_table):
    mesh = plsc.VectorSubcoreMesh(
        core_axis_name="c", subcore_axis_name="s",
        num_cores=_NC, num_subcores=_NS)
    gather_run = pl.kernel(
        _gather_body,
        out_type=(jax.ShapeDtypeStruct((_STG,), jnp.float32),
                  jax.ShapeDtypeStruct((_STG,), jnp.float32)),
        mesh=mesh,
        scratch_types=[
            pltpu.VMEM((4096,), jnp.int32),
            pltpu.VMEM((_CAP + _L,), jnp.int32),
            pltpu.VMEM((_CAP + _L,), jnp.int32),
            pltpu.VMEM((_CAP + _L,), jnp.int32),
            pltpu.VMEM((_CAP + _L,), jnp.int32),
            pltpu.VMEM((_WW,), jnp.float32),
            pltpu.VMEM((_WW,), jnp.float32),
            pltpu.VMEM((_CAP * _STRIDE,), jnp.float32),
            pltpu.SemaphoreType.DMA,
            pltpu.SemaphoreType.DMA,
            pltpu.SemaphoreType.DMA,
        ],
        compiler_params=pltpu.CompilerParams(needs_layout_passes=False),
    )
    dot_run = pl.kernel(
        _dot_body,
        out_type=jax.ShapeDtypeStruct((BATCH,), jnp.float32),
        mesh=mesh,
        scratch_types=[
            pltpu.VMEM((_BPT2 * _STRIDE,), jnp.float32),
            pltpu.VMEM((_BPT2 * _STRIDE,), jnp.float32),
            pltpu.VMEM((_BPT2,), jnp.float32),
            pltpu.SemaphoreType.DMA,
        ],
        compiler_params=pltpu.CompilerParams(needs_layout_passes=False),
    )
    utail = jnp.pad(user_table[_REM1:], ((0, 128 - (ROWS - _REM1)), (0, 0))).T
    itail = jnp.pad(item_table[_REM1:], ((0, 128 - (ROWS - _REM1)), (0, 0))).T
    ug, ig = gather_run(user_ids, item_ids, user_table.T, item_table.T,
                        utail, itail)
    return dot_run(ug, ig)


# speedup vs baseline: 1.1153x; 1.0083x over previous
import functools
import jax
import jax.numpy as jnp
from jax import lax
from jax.experimental import pallas as pl
from jax.experimental.pallas import tpu as pltpu
from jax.experimental.pallas import tpu_sc as plsc

BATCH = 16384
EMBED_DIM = 64
ROWS = 1000001

_NC = 2
_NS = 16
_L = 16
_NW = _NC * _NS

_REM1 = (ROWS // 128) * 128
_WT = 245
_WW = _WT * 128
_WWL = _WW - 256
_LOMAX = _REM1 - _WWL
_CAP = 768
_SENT = BATCH
_STRIDE = 72
_STG = (BATCH + 1) * _STRIDE


def _gather_body(uids, iids, utab, itab, utail, itail, ug, ig,
                 idbuf, cr_u, cb_u, cr_i, cb_i, slab_a, slab_b, rows,
                 sem_a, sem_b, esem):
    cid = lax.axis_index("c")
    sid = lax.axis_index("s")
    wid = sid * _NC + cid
    lanes = lax.iota(jnp.int32, _L)
    lo = jnp.minimum(wid * _WW, _LOMAX)
    hi = lo + _WW
    last = wid == _NW - 1

    def extract(ids_hbm, cr, cb):
        def chunk(c, ou):
            pltpu.sync_copy(
                ids_hbm.at[pl.ds(pl.multiple_of(c * 4096, 4096), 4096)],
                idbuf)

            def vec(i, ou):
                r = idbuf[pl.ds(pl.multiple_of(i * _L, _L), _L)] + 1
                b = c * 4096 + i * _L + lanes
                m = (r >= lo) & (r < hi)
                cnt = plsc.all_reduce_population_count(m)[0]
                plsc.store_compressed(cr.at[pl.ds(ou, _L)], r - lo, mask=m)
                plsc.store_compressed(cb.at[pl.ds(ou, _L)], b, mask=m)
                return jnp.minimum(ou + cnt, _CAP)

            return lax.fori_loop(0, 4096 // _L, vec, ou)

        ou = lax.fori_loop(0, BATCH // 4096, chunk, jnp.int32(0))
        full = jnp.ones((_L,), jnp.bool_)
        plsc.store_compressed(cr.at[pl.ds(ou, _L)],
                              jnp.zeros((_L,), jnp.int32), mask=full)
        plsc.store_compressed(cb.at[pl.ds(ou, _L)],
                              jnp.full((_L,), _SENT, jnp.int32), mask=full)
        return (ou + _L - 1) // _L

    def stage(tab, tail, slab, d, s):
        @pl.when(jnp.logical_not(last))
        def _main():
            pltpu.async_copy(tab.at[d, pl.ds(pl.multiple_of(lo, 128), _WW)],
                             slab, s)

        @pl.when(last)
        def _last():
            pltpu.async_copy(
                tab.at[d, pl.ds(pl.multiple_of(lo, 128), _WWL)],
                slab.at[pl.ds(0, _WWL)], s)
            pltpu.async_copy(tail.at[d], slab.at[pl.ds(_WWL, 128)], s)

    def stage_wait(tab, tail, slab, d, s):
        @pl.when(jnp.logical_not(last))
        def _main():
            pltpu.make_async_copy(
                tab.at[d, pl.ds(pl.multiple_of(lo, 128), _WW)],
                slab, s).wait()

        @pl.when(last)
        def _last():
            pltpu.make_async_copy(
                tab.at[d, pl.ds(pl.multiple_of(lo, 128), _WWL)],
                slab.at[pl.ds(0, _WWL)], s).wait()
            pltpu.make_async_copy(tail.at[d], slab.at[pl.ds(_WWL, 128)],
                                  s).wait()

    def table_phase(tab, tail, cr, cb, ng, out_hbm, nxt, prolog=True):
        def gather_d(slab, d):
            def grp(g, _):
                rl = cr[pl.ds(pl.multiple_of(g * _L, _L), _L)]
                vals = plsc.load_gather(slab, [rl])
                plsc.store_scatter(rows,
                                   [(g * _L + lanes) * _STRIDE + d], vals)
                return 0

            lax.fori_loop(0, ng, grp, 0)

        if prolog:
            stage(tab, tail, slab_a, 0, sem_a)

        def pair(dp, _):
            d0 = dp * 2
            stage_wait(tab, tail, slab_a, d0, sem_a)
            stage(tab, tail, slab_b, d0 + 1, sem_b)
            gather_d(slab_a, d0)
            stage_wait(tab, tail, slab_b, d0 + 1, sem_b)

            @pl.when(dp + 1 < EMBED_DIM // 2)
            def _pf():
                stage(tab, tail, slab_a, d0 + 2, sem_a)

            gather_d(slab_b, d0 + 1)
            return 0

        lax.fori_loop(0, EMBED_DIM // 2, pair, 0)
        nxt()

        def egrp(g, _):
            bv = cb[pl.ds(pl.multiple_of(g * _L, _L), _L)]
            for l in range(_L):
                k = g * _L + l
                src = rows.at[pl.ds(k * _STRIDE, EMBED_DIM)]
                dst = out_hbm.at[pl.ds(pl.multiple_of(bv[l] * _STRIDE, 8),
                                       EMBED_DIM)]
                pltpu.async_copy(src, dst, esem)
            return 0

        lax.fori_loop(0, ng, egrp, 0)

        def edrain(k, _):
            pltpu.make_async_copy(
                rows.at[pl.ds(0, EMBED_DIM)],
                out_hbm.at[pl.ds(_SENT * _STRIDE, EMBED_DIM)], esem).wait()
            return 0

        lax.fori_loop(0, ng * _L, edrain, 0)

    stage(utab, utail, slab_a, 0, sem_a)
    ng_u = extract(uids, cr_u, cb_u)
    ng_i = extract(iids, cr_i, cb_i)
    table_phase(utab, utail, cr_u, cb_u, ng_u, ug,
                lambda: stage(itab, itail, slab_a, 0, sem_a), prolog=False)
    table_phase(itab, itail, cr_i, cb_i, ng_i, ig, lambda: None,
                prolog=False)


_BPT2 = BATCH // _NW
_CH2 = 256


def _dot_body(ug, ig, out, us, vs, outv, sem):
    cid = lax.axis_index("c")
    sid = lax.axis_index("s")
    wid = sid * _NC + cid
    base = wid * _BPT2
    lanes = lax.iota(jnp.int32, _L)

    b0 = pl.multiple_of(base * _STRIDE, 8)
    pltpu.sync_copy(ug.at[pl.ds(b0, _BPT2 * _STRIDE)], us)
    pltpu.sync_copy(ig.at[pl.ds(b0, _BPT2 * _STRIDE)], vs)

    def grp(g, _):
        flat = (g * _L + lanes) * _STRIDE
        acc = jnp.zeros((_L,), jnp.float32)
        for d in range(EMBED_DIM):
            u = plsc.load_gather(us, [flat + d])
            v = plsc.load_gather(vs, [flat + d])
            acc = acc + u * v
        outv[pl.ds(pl.multiple_of(g * _L, _L), _L)] = acc
        return 0

    lax.fori_loop(0, _BPT2 // _L, grp, 0)

    pltpu.sync_copy(outv, out.at[pl.ds(pl.multiple_of(base, 8), _BPT2)])


@jax.jit
def kernel(user_ids, item_ids, user_table, item_table):
    mesh = plsc.VectorSubcoreMesh(
        core_axis_name="c", subcore_axis_name="s",
        num_cores=_NC, num_subcores=_NS)
    gather_run = pl.kernel(
        _gather_body,
        out_type=(jax.ShapeDtypeStruct((_STG,), jnp.float32),
                  jax.ShapeDtypeStruct((_STG,), jnp.float32)),
        mesh=mesh,
        scratch_types=[
            pltpu.VMEM((4096,), jnp.int32),
            pltpu.VMEM((_CAP + _L,), jnp.int32),
            pltpu.VMEM((_CAP + _L,), jnp.int32),
            pltpu.VMEM((_CAP + _L,), jnp.int32),
            pltpu.VMEM((_CAP + _L,), jnp.int32),
            pltpu.VMEM((_WW,), jnp.float32),
            pltpu.VMEM((_WW,), jnp.float32),
            pltpu.VMEM((_CAP * _STRIDE,), jnp.float32),
            pltpu.SemaphoreType.DMA,
            pltpu.SemaphoreType.DMA,
            pltpu.SemaphoreType.DMA,
        ],
        compiler_params=pltpu.CompilerParams(needs_layout_passes=False),
    )
    dot_run = pl.kernel(
        _dot_body,
        out_type=jax.ShapeDtypeStruct((BATCH,), jnp.float32),
        mesh=mesh,
        scratch_types=[
            pltpu.VMEM((_BPT2 * _STRIDE,), jnp.float32),
            pltpu.VMEM((_BPT2 * _STRIDE,), jnp.float32),
            pltpu.VMEM((_BPT2,), jnp.float32),
            pltpu.SemaphoreType.DMA,
        ],
        compiler_params=pltpu.CompilerParams(needs_layout_passes=False),
    )
    utail = jnp.pad(user_table[_REM1:], ((0, 128 - (ROWS - _REM1)), (0, 0))).T
    itail = jnp.pad(item_table[_REM1:], ((0, 128 - (ROWS - _REM1)), (0, 0))).T
    ug, ig = gather_run(user_ids, item_ids, user_table.T, item_table.T,
                        utail, itail)
    return dot_run(ug, ig)


# speedup vs baseline: 1.1174x; 1.0019x over previous
import jax
import jax.numpy as jnp
from jax import lax
from jax.experimental import pallas as pl
from jax.experimental.pallas import tpu as pltpu
from jax.experimental.pallas import tpu_sc as plsc

BATCH = 16384
EMBED_DIM = 64
ROWS = 1000001

_NC = 2
_NS = 16
_L = 16
_NW = _NC * _NS

_REM1 = (ROWS // 128) * 128
_WT = 245
_WW = _WT * 128
_WWL = _WW - 256
_LOMAX = _REM1 - _WWL
_CAP = 768
_SENT = BATCH
_STRIDE = 72
_STG = (BATCH + 1) * _STRIDE


def _gather_body(uids, iids, utab, itab, utail, itail, ug, ig,
                 idbuf, cr_u, cb_u, cr_i, cb_i, slab_a, slab_b, rows,
                 sem_a, sem_b, esem):
    cid = lax.axis_index("c")
    sid = lax.axis_index("s")
    wid = sid * _NC + cid
    lanes = lax.iota(jnp.int32, _L)
    lo = jnp.minimum(wid * _WW, _LOMAX)
    hi = lo + _WW
    last = wid == _NW - 1

    def extract(ids_hbm, cr, cb):
        def chunk(c, ou):
            pltpu.sync_copy(
                ids_hbm.at[pl.ds(pl.multiple_of(c * 4096, 4096), 4096)],
                idbuf)

            def vec(i, ou):
                r = idbuf[pl.ds(pl.multiple_of(i * _L, _L), _L)] + 1
                b = c * 4096 + i * _L + lanes
                m = (r >= lo) & (r < hi)
                cnt = plsc.all_reduce_population_count(m)[0]
                plsc.store_compressed(cr.at[pl.ds(ou, _L)], r - lo, mask=m)
                plsc.store_compressed(cb.at[pl.ds(ou, _L)], b, mask=m)
                return jnp.minimum(ou + cnt, _CAP)

            return lax.fori_loop(0, 4096 // _L, vec, ou)

        ou = lax.fori_loop(0, BATCH // 4096, chunk, jnp.int32(0))
        full = jnp.ones((_L,), jnp.bool_)
        plsc.store_compressed(cr.at[pl.ds(ou, _L)],
                              jnp.zeros((_L,), jnp.int32), mask=full)
        plsc.store_compressed(cb.at[pl.ds(ou, _L)],
                              jnp.full((_L,), _SENT, jnp.int32), mask=full)
        return (ou + _L - 1) // _L

    def stage(tab, tail, slab, d, s):
        @pl.when(jnp.logical_not(last))
        def _main():
            pltpu.async_copy(tab.at[d, pl.ds(pl.multiple_of(lo, 128), _WW)],
                             slab, s)

        @pl.when(last)
        def _last():
            pltpu.async_copy(
                tab.at[d, pl.ds(pl.multiple_of(lo, 128), _WWL)],
                slab.at[pl.ds(0, _WWL)], s)
            pltpu.async_copy(tail.at[d], slab.at[pl.ds(_WWL, 128)], s)

    def stage_wait(tab, tail, slab, d, s):
        @pl.when(jnp.logical_not(last))
        def _main():
            pltpu.make_async_copy(
                tab.at[d, pl.ds(pl.multiple_of(lo, 128), _WW)],
                slab, s).wait()

        @pl.when(last)
        def _last():
            pltpu.make_async_copy(
                tab.at[d, pl.ds(pl.multiple_of(lo, 128), _WWL)],
                slab.at[pl.ds(0, _WWL)], s).wait()
            pltpu.make_async_copy(tail.at[d], slab.at[pl.ds(_WWL, 128)],
                                  s).wait()

    def table_phase(tab, tail, cr, cb, ng, out_hbm, nxt, prolog=True):
        def gather_d(slab, d):
            def grp(g, _):
                rl = cr[pl.ds(pl.multiple_of(g * _L, _L), _L)]
                vals = plsc.load_gather(slab, [rl])
                plsc.store_scatter(rows,
                                   [(g * _L + lanes) * _STRIDE + d], vals)
                return 0

            lax.fori_loop(0, ng, grp, 0)

        if prolog:
            stage(tab, tail, slab_a, 0, sem_a)

        def pair(dp, _):
            d0 = dp * 2
            stage_wait(tab, tail, slab_a, d0, sem_a)
            stage(tab, tail, slab_b, d0 + 1, sem_b)
            gather_d(slab_a, d0)
            stage_wait(tab, tail, slab_b, d0 + 1, sem_b)

            @pl.when(dp + 1 < EMBED_DIM // 2)
            def _pf():
                stage(tab, tail, slab_a, d0 + 2, sem_a)

            gather_d(slab_b, d0 + 1)
            return 0

        lax.fori_loop(0, EMBED_DIM // 2, pair, 0)
        nxt()

        def egrp(g, _):
            bv = cb[pl.ds(pl.multiple_of(g * _L, _L), _L)]
            for l in range(_L):
                k = g * _L + l
                src = rows.at[pl.ds(k * _STRIDE, EMBED_DIM)]
                dst = out_hbm.at[pl.ds(pl.multiple_of(bv[l] * _STRIDE, 8),
                                       EMBED_DIM)]
                pltpu.async_copy(src, dst, esem)
            return 0

        lax.fori_loop(0, ng, egrp, 0)

        def edrain(k, _):
            pltpu.make_async_copy(
                rows.at[pl.ds(0, EMBED_DIM)],
                out_hbm.at[pl.ds(_SENT * _STRIDE, EMBED_DIM)], esem).wait()
            return 0

        lax.fori_loop(0, ng * _L, edrain, 0)

    stage(utab, utail, slab_a, 0, sem_a)
    ng_u = extract(uids, cr_u, cb_u)
    ng_i = extract(iids, cr_i, cb_i)
    table_phase(utab, utail, cr_u, cb_u, ng_u, ug,
                lambda: stage(itab, itail, slab_a, 0, sem_a), prolog=False)
    table_phase(itab, itail, cr_i, cb_i, ng_i, ig, lambda: None,
                prolog=False)


_BPT2 = BATCH // _NW


def _dot_body(ug, ig, out, us, vs, outv, sem):
    cid = lax.axis_index("c")
    sid = lax.axis_index("s")
    wid = sid * _NC + cid
    base = wid * _BPT2
    lanes = lax.iota(jnp.int32, _L)

    b0 = pl.multiple_of(base * _STRIDE, 8)
    pltpu.sync_copy(ug.at[pl.ds(b0, _BPT2 * _STRIDE)], us)
    pltpu.sync_copy(ig.at[pl.ds(b0, _BPT2 * _STRIDE)], vs)

    def grp(g, _):
        flat = (g * _L + lanes) * _STRIDE
        acc = jnp.zeros((_L,), jnp.float32)
        for d in range(EMBED_DIM):
            u = plsc.load_gather(us, [flat + d])
            v = plsc.load_gather(vs, [flat + d])
            acc = acc + u * v
        outv[pl.ds(pl.multiple_of(g * _L, _L), _L)] = acc
        return 0

    lax.fori_loop(0, _BPT2 // _L, grp, 0)

    pltpu.sync_copy(outv, out.at[pl.ds(pl.multiple_of(base, 8), _BPT2)])


@jax.jit
def kernel(user_ids, item_ids, user_table, item_table):
    mesh = plsc.VectorSubcoreMesh(
        core_axis_name="c", subcore_axis_name="s",
        num_cores=_NC, num_subcores=_NS)
    gather_run = pl.kernel(
        _gather_body,
        out_type=(jax.ShapeDtypeStruct((_STG,), jnp.float32),
                  jax.ShapeDtypeStruct((_STG,), jnp.float32)),
        mesh=mesh,
        scratch_types=[
            pltpu.VMEM((4096,), jnp.int32),
            pltpu.VMEM((_CAP + _L,), jnp.int32),
            pltpu.VMEM((_CAP + _L,), jnp.int32),
            pltpu.VMEM((_CAP + _L,), jnp.int32),
            pltpu.VMEM((_CAP + _L,), jnp.int32),
            pltpu.VMEM((_WW,), jnp.float32),
            pltpu.VMEM((_WW,), jnp.float32),
            pltpu.VMEM((_CAP * _STRIDE,), jnp.float32),
            pltpu.SemaphoreType.DMA,
            pltpu.SemaphoreType.DMA,
            pltpu.SemaphoreType.DMA,
        ],
        compiler_params=pltpu.CompilerParams(needs_layout_passes=False),
    )
    dot_run = pl.kernel(
        _dot_body,
        out_type=jax.ShapeDtypeStruct((BATCH,), jnp.float32),
        mesh=mesh,
        scratch_types=[
            pltpu.VMEM((_BPT2 * _STRIDE,), jnp.float32),
            pltpu.VMEM((_BPT2 * _STRIDE,), jnp.float32),
            pltpu.VMEM((_BPT2,), jnp.float32),
            pltpu.SemaphoreType.DMA,
        ],
        compiler_params=pltpu.CompilerParams(needs_layout_passes=False),
    )
    utail = jnp.pad(user_table[_REM1:], ((0, 128 - (ROWS - _REM1)), (0, 0))).T
    itail = jnp.pad(item_table[_REM1:], ((0, 128 - (ROWS - _REM1)), (0, 0))).T
    ug, ig = gather_run(user_ids, item_ids, user_table.T, item_table.T,
                        utail, itail)
    return dot_run(ug, ig)
